# baseline (speedup 1.0000x reference)
import functools

import jax
import jax.numpy as jnp
from jax import lax
from jax.experimental import pallas as pl
from jax.experimental.pallas import tpu as pltpu

_N_CLS = 100
_CTX_LEN = 77
_N_CTX = 4
_PREFIX = 4
_EMBED = 512
_ZEROS = _CTX_LEN - _PREFIX - _N_CTX - 1
_WAVE = 10


def _tc_full(table, ctx, idx8):
    def body(idx_ref, table_ref, ctx_ref, out_ref, prompt, gsem, osem):
        for i in range(_PREFIX):
            pltpu.make_async_copy(
                table_ref.at[pl.ds(idx_ref[i], 1)], prompt.at[pl.ds(i, 1)], gsem
            ).start()
        pltpu.make_async_copy(
            table_ref.at[pl.ds(idx_ref[_PREFIX], 1)],
            prompt.at[pl.ds(_CTX_LEN - 1, 1)],
            gsem,
        ).start()
        prompt[pl.ds(_PREFIX, _N_CTX), :] = ctx_ref[...]
        prompt[pl.ds(_PREFIX + _N_CTX, _ZEROS), :] = jnp.zeros(
            (_ZEROS, _EMBED), jnp.float32
        )
        for i in range(_PREFIX):
            pltpu.make_async_copy(
                table_ref.at[pl.ds(idx_ref[i], 1)], prompt.at[pl.ds(i, 1)], gsem
            ).wait()
        pltpu.make_async_copy(
            table_ref.at[pl.ds(idx_ref[_PREFIX], 1)],
            prompt.at[pl.ds(_CTX_LEN - 1, 1)],
            gsem,
        ).wait()
        for base in range(0, _N_CLS, _WAVE):
            for c in range(base, base + _WAVE):
                pltpu.make_async_copy(
                    prompt, out_ref.at[c], osem.at[c - base]
                ).start()
            for c in range(base, base + _WAVE):
                pltpu.make_async_copy(
                    prompt, out_ref.at[c], osem.at[c - base]
                ).wait()

    grid_spec = pltpu.PrefetchScalarGridSpec(
        num_scalar_prefetch=1,
        grid=(1,),
        in_specs=[
            pl.BlockSpec(memory_space=pl.ANY),
            pl.BlockSpec(memory_space=pltpu.VMEM),
        ],
        out_specs=pl.BlockSpec(memory_space=pl.ANY),
        scratch_shapes=[
            pltpu.VMEM((_CTX_LEN, _EMBED), jnp.float32),
            pltpu.SemaphoreType.DMA,
            pltpu.SemaphoreType.DMA((_WAVE,)),
        ],
    )
    return pl.pallas_call(
        body,
        grid_spec=grid_spec,
        out_shape=jax.ShapeDtypeStruct((_N_CLS, _CTX_LEN, _EMBED), jnp.float32),
    )(idx8, table, ctx)


def kernel(token_embedding, ctx_vectors, tokenized_prompt):
    idx8 = jnp.concatenate(
        [
            tokenized_prompt[:_PREFIX],
            tokenized_prompt[_CTX_LEN - 1 :],
            jnp.zeros((3,), jnp.int32),
        ]
    )
    return _tc_full(token_embedding, ctx_vectors, idx8)
